# baseline (speedup 1.0000x reference)
import functools

import jax
import jax.numpy as jnp
from jax import lax
from jax.experimental import pallas as pl
from jax.experimental.pallas import tpu as pltpu
from jax.experimental.pallas import tpu_sc as plsc

BATCH = 16384
D = 32
NT = 4
NC = 2
NS = 16
NW = NC * NS
BPW = BATCH // NW
CHUNK = 128
NCHUNK = BPW // CHUNK

_MESH = plsc.VectorSubcoreMesh(core_axis_name="c", subcore_axis_name="s")


@functools.partial(
    pl.kernel,
    out_type=jax.ShapeDtypeStruct((BATCH, NT * D), jnp.float32),
    mesh=_MESH,
    compiler_params=pltpu.CompilerParams(use_tc_tiling_on_sc=False),
    scratch_types=[
        pltpu.VMEM((NT, NCHUNK, CHUNK), jnp.int32),
        pltpu.VMEM((NT, BPW, D), jnp.float32),
        pltpu.SemaphoreType.DMA,
    ],
)
def _emb_kernel(idx_hbm, w_gender, w_age, w_occ, w_area, out_hbm,
                idx_v, rows_v, gsem):
    wid = lax.axis_index("s") * NC + lax.axis_index("c")
    base = wid * BPW
    pltpu.sync_copy(idx_hbm.at[wid], idx_v)
    tables = (w_gender, w_age, w_occ, w_area)
    copies = []
    for t in range(NT):
        for j in range(NCHUNK):
            copies.append(pltpu.async_copy(
                tables[t].at[idx_v.at[t, j]],
                rows_v.at[t, pl.ds(j * CHUNK, CHUNK)],
                gsem,
            ))
    for c in copies:
        c.wait()
    for t in range(NT):
        pltpu.sync_copy(
            rows_v.at[t],
            out_hbm.at[pl.ds(base, BPW), pl.ds(t * D, D)],
        )


def kernel(gender_idx, age_idx, occupation_idx, area_idx,
           W_gender, W_age, W_occupation, W_area):
    idx_all = jnp.stack(
        [gender_idx.astype(jnp.int32), age_idx.astype(jnp.int32),
         occupation_idx.astype(jnp.int32), area_idx.astype(jnp.int32)],
        axis=0,
    )
    idx_all = idx_all.reshape(NT, NW, NCHUNK, CHUNK).transpose(1, 0, 2, 3)
    return _emb_kernel(idx_all, W_gender, W_age, W_occupation, W_area)


# speedup vs baseline: 2.4726x; 2.4726x over previous
import functools

import jax
import jax.numpy as jnp
from jax import lax
from jax.experimental import pallas as pl
from jax.experimental.pallas import tpu as pltpu
from jax.experimental.pallas import tpu_sc as plsc

BATCH = 16384
D = 32
NT = 4
NC = 2
NS = 16
NW = NC * NS
BPW = BATCH // NW
CHUNK = 128
NCHUNK = BPW // CHUNK
L = 16
GROUPS = BPW // L

_MESH = plsc.VectorSubcoreMesh(core_axis_name="c", subcore_axis_name="s")


@functools.partial(
    pl.kernel,
    out_type=jax.ShapeDtypeStruct((BATCH, NT * D), jnp.float32),
    mesh=_MESH,
    compiler_params=pltpu.CompilerParams(use_tc_tiling_on_sc=False, needs_layout_passes=False),
    scratch_types=[
        pltpu.VMEM((NT, BPW), jnp.int32),
        pltpu.VMEM((2 * L, D), jnp.float32),
        pltpu.VMEM((BPW, D), jnp.float32),
        pltpu.VMEM((BPW, (NT - 1) * D), jnp.float32),
        pltpu.SemaphoreType.DMA,
        pltpu.SemaphoreType.DMA,
    ],
)
def _emb_kernel(idx_hbm, w_gender, w_age, w_occ, w_area, out_hbm,
                idx_v, small_v, area_v, big_v, gsem, osem):
    wid = lax.axis_index("s") * NC + lax.axis_index("c")
    base = wid * BPW
    pltpu.sync_copy(idx_hbm.at[wid], idx_v)
    copies = [
        pltpu.async_copy(
            w_area.at[idx_v.at[3, pl.ds(j * CHUNK, CHUNK)]],
            area_v.at[pl.ds(j * CHUNK, CHUNK)],
            gsem,
        )
        for j in range(NCHUNK)
    ]
    pltpu.sync_copy(w_gender, small_v.at[pl.ds(0, 2)])
    pltpu.sync_copy(w_age, small_v.at[pl.ds(2, 7)])
    pltpu.sync_copy(w_occ, small_v.at[pl.ds(9, 21)])

    lane = lax.broadcasted_iota(jnp.int32, (L,), 0)

    def group_body(i, carry):
        rbase = i * L
        rows = rbase + lane
        for t in range(NT - 1):
            ridx = idx_v[t, pl.ds(rbase, L)]
            for c in range(D):
                col = jnp.full((L,), c, jnp.int32)
                val = plsc.load_gather(small_v, [ridx, col])
                outcol = jnp.full((L,), t * D + c, jnp.int32)
                plsc.store_scatter(big_v, [rows, outcol], val)
        return carry

    lax.fori_loop(0, GROUPS, group_body, 0)
    wr_small = pltpu.async_copy(
        big_v, out_hbm.at[pl.ds(base, BPW), pl.ds(0, (NT - 1) * D)], osem)
    for cpy in copies:
        cpy.wait()
    wr_area = pltpu.async_copy(
        area_v, out_hbm.at[pl.ds(base, BPW), pl.ds((NT - 1) * D, D)], osem)
    wr_small.wait()
    wr_area.wait()


def kernel(gender_idx, age_idx, occupation_idx, area_idx,
           W_gender, W_age, W_occupation, W_area):
    idx_all = jnp.stack(
        [gender_idx.astype(jnp.int32),
         age_idx.astype(jnp.int32) + 2,
         occupation_idx.astype(jnp.int32) + 9,
         area_idx.astype(jnp.int32)],
        axis=0,
    )
    idx_all = idx_all.reshape(NT, NW, BPW).transpose(1, 0, 2)
    return _emb_kernel(idx_all, W_gender, W_age, W_occupation, W_area)
